# Initial kernel scaffold; baseline (speedup 1.0000x reference)
import jax
import jax.numpy as jnp
from jax.experimental import pallas as pl


def kernel(x, edge_index, batch, gn0_weight, gn0_bias, gn0_mean_scale, W1, b1, gn1_weight, gn1_bias, gn1_mean_scale, W2, b2, gn2_weight, gn2_bias, gn2_mean_scale, W3, b3, Wd, bd, Wo, bo):
    raise NotImplementedError("write your pallas kernel here")


# speedup vs baseline: 24.4918x; 24.4918x over previous
import functools

import jax
import jax.numpy as jnp
from jax import lax
from jax.experimental import pallas as pl
from jax.experimental.pallas import tpu as pltpu
from jax.experimental.pallas import tpu_sc as plsc

N = 10000
E = 320000
F_IN = 128
HID = 64
C = 10
EPS = 1e-5

NC = 2
NS = 16
NW = NC * NS
CHUNK = 128
CH = 80
E_PAD = NW * CH * CHUNK
N_PAD = 10240
RPT = N_PAD // NS
ZB = RPT // CHUNK
DW = 16

_mesh = plsc.VectorSubcoreMesh(core_axis_name="c", subcore_axis_name="s")
_sc_params = pltpu.CompilerParams(use_tc_tiling_on_sc=False)


@functools.partial(
    pl.kernel,
    out_type=jax.ShapeDtypeStruct((NC, N_PAD, DW), jnp.float32),
    mesh=_mesh,
    compiler_params=_sc_params,
    scratch_types=[
        pltpu.VMEM((CH, CHUNK), jnp.int32),
        pltpu.VMEM((CHUNK, DW), jnp.float32),
        pltpu.VMEM((CHUNK, DW), jnp.float32),
        pltpu.VMEM_SHARED((N_PAD, DW), jnp.float32),
    ],
)
def _sc_deg(dst_hbm, out_hbm, dst_v, ones_v, zbuf, agg):
    cid = lax.axis_index("c")
    sid = lax.axis_index("s")
    wid = cid * NS + sid

    lane = lax.iota(jnp.int32, 16)
    unit = jnp.where(lane == 0, 1.0, 0.0)
    zero = jnp.zeros((16,), jnp.float32)

    def _fill(i, _):
        ones_v[i, pl.ds(0, 16)] = unit
        zbuf[i, pl.ds(0, 16)] = zero
        return 0

    lax.fori_loop(0, CHUNK, _fill, 0)

    for t in range(ZB):
        pltpu.sync_copy(zbuf, agg.at[pl.ds((sid * ZB + t) * CHUNK, CHUNK)])
    plsc.subcore_barrier()

    pltpu.sync_copy(dst_hbm.at[wid], dst_v)

    def _chunk(j, _):
        pltpu.sync_copy(ones_v, agg.at[dst_v.at[j]], add=True)
        return 0

    lax.fori_loop(0, CH, _chunk, 0)
    plsc.subcore_barrier()

    for t in range(ZB):
        r0 = (sid * ZB + t) * CHUNK
        pltpu.sync_copy(agg.at[pl.ds(r0, CHUNK)], zbuf)
        pltpu.sync_copy(zbuf, out_hbm.at[cid, pl.ds(r0, CHUNK)])


@functools.partial(
    pl.kernel,
    out_type=jax.ShapeDtypeStruct((NC, N_PAD, HID), jnp.float32),
    mesh=_mesh,
    compiler_params=_sc_params,
    scratch_types=[
        pltpu.VMEM((CH, CHUNK), jnp.int32),
        pltpu.VMEM((CH, CHUNK), jnp.int32),
        pltpu.VMEM((CHUNK, HID), jnp.float32),
        pltpu.VMEM((CHUNK, HID), jnp.float32),
        pltpu.VMEM_SHARED((N_PAD, HID), jnp.float32),
        pltpu.SemaphoreType.DMA,
    ],
)
def _sc_conv(src_hbm, dst_hbm, p_hbm, out_hbm, src_v, dst_v, rows_v, zbuf,
             agg, sem):
    cid = lax.axis_index("c")
    sid = lax.axis_index("s")
    wid = cid * NS + sid

    zero = jnp.zeros((16,), jnp.float32)

    def _fill(i, _):
        for c0 in range(HID // 16):
            zbuf[i, pl.ds(c0 * 16, 16)] = zero
        return 0

    lax.fori_loop(0, CHUNK, _fill, 0)

    for t in range(ZB):
        pltpu.sync_copy(zbuf, agg.at[pl.ds((sid * ZB + t) * CHUNK, CHUNK)])
    plsc.subcore_barrier()

    pltpu.sync_copy(src_hbm.at[wid], src_v)
    pltpu.sync_copy(dst_hbm.at[wid], dst_v)

    def _chunk(j, _):
        pltpu.async_copy(p_hbm.at[src_v.at[j]], rows_v, sem).wait()
        pltpu.sync_copy(rows_v, agg.at[dst_v.at[j]], add=True)
        return 0

    lax.fori_loop(0, CH, _chunk, 0)
    plsc.subcore_barrier()

    for t in range(ZB):
        r0 = (sid * ZB + t) * CHUNK
        pltpu.sync_copy(agg.at[pl.ds(r0, CHUNK)], zbuf)
        pltpu.sync_copy(zbuf, out_hbm.at[cid, pl.ds(r0, CHUNK)])


def _dinv(degr):
    hist = degr[0, :N, 0:1] + degr[1, :N, 0:1]
    return lax.rsqrt(hist + 1.0)


def _pad_cols(g):
    return jnp.concatenate([g, jnp.zeros((N, 128 - HID), jnp.float32)],
                           axis=1)


def _tca(x_ref, gw, gb, gms, w1, degr, p_ref):
    x = x_ref[...]
    mean = jnp.sum(x, axis=0, keepdims=True) * (1.0 / N)
    xc = x - gms[...] * mean
    var = jnp.sum(xc * xc, axis=0, keepdims=True) * (1.0 / N)
    xn = gw[...] * xc * lax.rsqrt(var + EPS) + gb[...]
    g = jnp.dot(xn, w1[...], preferred_element_type=jnp.float32)
    p_ref[...] = _pad_cols(g * _dinv(degr))


def _tcmid(s_ref, p_ref, b_r, gw, gb, gms, w_r, degr, out_ref):
    dinv = _dinv(degr)
    s = s_ref[0, :N, :] + s_ref[1, :N, :]
    c = dinv * (s + p_ref[:, :HID]) + b_r[...]
    a = jnp.maximum(c, 0.0)
    mean = jnp.sum(a, axis=0, keepdims=True) * (1.0 / N)
    ac = a - gms[...] * mean
    var = jnp.sum(ac * ac, axis=0, keepdims=True) * (1.0 / N)
    h = gw[...] * ac * lax.rsqrt(var + EPS) + gb[...]
    g = jnp.dot(h, w_r[...], preferred_element_type=jnp.float32)
    out_ref[...] = _pad_cols(g * dinv)


def _tcd(s_ref, p_ref, b_r, degr, wd_r, bd_r, wo_r, bo_r, out_ref):
    dinv = _dinv(degr)
    s = s_ref[0, :N, :] + s_ref[1, :N, :]
    c = dinv * (s + p_ref[:, :HID]) + b_r[...]
    a = jnp.maximum(c, 0.0)
    pooled = jnp.sum(a, axis=0, keepdims=True) * (1.0 / N)
    d = jnp.maximum(
        jnp.dot(pooled, wd_r[...], preferred_element_type=jnp.float32)
        + bd_r[...], 0.0)
    logits = (jnp.dot(d, wo_r[...], preferred_element_type=jnp.float32)
              + bo_r[...])
    m = jnp.max(logits, axis=1, keepdims=True)
    e = jnp.exp(logits - m)
    out_ref[...] = e / jnp.sum(e, axis=1, keepdims=True)


def _tc(body, out_shape, *args):
    return pl.pallas_call(
        body, out_shape=jax.ShapeDtypeStruct(out_shape, jnp.float32))(*args)


def kernel(x, edge_index, batch, gn0_weight, gn0_bias, gn0_mean_scale,
           W1, b1, gn1_weight, gn1_bias, gn1_mean_scale, W2, b2,
           gn2_weight, gn2_bias, gn2_mean_scale, W3, b3, Wd, bd, Wo, bo):
    src = edge_index[0].astype(jnp.int32)
    dst = edge_index[1].astype(jnp.int32)
    pad = E_PAD - E
    pad_src = (jnp.arange(pad, dtype=jnp.int32) * 97) % N
    pad_dst = N + (jnp.arange(pad, dtype=jnp.int32) % (N_PAD - N))
    src_p = (2 * jnp.concatenate([src, pad_src])).reshape(NW, CH, CHUNK)
    dst_p = jnp.concatenate([dst, pad_dst]).reshape(NW, CH, CHUNK)

    row = lambda v: v.reshape(1, -1)

    deg = _sc_deg(dst_p)

    p1 = _tc(_tca, (N, 128), x, row(gn0_weight), row(gn0_bias),
             row(gn0_mean_scale), W1, deg)
    s1 = _sc_conv(src_p, dst_p, p1.reshape(2 * N, HID))
    p2 = _tc(_tcmid, (N, 128), s1, p1, row(b1), row(gn1_weight),
             row(gn1_bias), row(gn1_mean_scale), W2, deg)
    s2 = _sc_conv(src_p, dst_p, p2.reshape(2 * N, HID))
    p3 = _tc(_tcmid, (N, 128), s2, p2, row(b2), row(gn2_weight),
             row(gn2_bias), row(gn2_mean_scale), W3, deg)
    s3 = _sc_conv(src_p, dst_p, p3.reshape(2 * N, HID))
    out = _tc(_tcd, (1, C), s3, p3, row(b3), deg, Wd, row(bd), Wo, row(bo))
    return out


# speedup vs baseline: 35.3892x; 1.4449x over previous
import functools

import jax
import jax.numpy as jnp
from jax import lax
from jax.experimental import pallas as pl
from jax.experimental.pallas import tpu as pltpu
from jax.experimental.pallas import tpu_sc as plsc

N = 10000
E = 320000
F_IN = 128
HID = 64
C = 10
EPS = 1e-5

NC = 2
NS = 16
NW = NC * NS
CHUNK = 128
CH = 80
E_PAD = NW * CH * CHUNK
N_PAD = 10240
RPT = N_PAD // NS
ZB = RPT // CHUNK
DW = 16

_mesh = plsc.VectorSubcoreMesh(core_axis_name="c", subcore_axis_name="s")
_sc_params = pltpu.CompilerParams(use_tc_tiling_on_sc=False)


@functools.partial(
    pl.kernel,
    out_type=jax.ShapeDtypeStruct((NC, N_PAD, DW), jnp.float32),
    mesh=_mesh,
    compiler_params=_sc_params,
    scratch_types=[
        pltpu.VMEM((CH, CHUNK), jnp.int32),
        pltpu.VMEM((CHUNK, DW), jnp.float32),
        pltpu.VMEM((CHUNK, DW), jnp.float32),
        pltpu.VMEM_SHARED((N_PAD, DW), jnp.float32),
    ],
)
def _sc_deg(dst_hbm, out_hbm, dst_v, ones_v, zbuf, agg):
    cid = lax.axis_index("c")
    sid = lax.axis_index("s")
    wid = cid * NS + sid

    lane = lax.iota(jnp.int32, 16)
    unit = jnp.where(lane == 0, 1.0, 0.0)
    zero = jnp.zeros((16,), jnp.float32)

    def _fill(i, _):
        ones_v[i, pl.ds(0, 16)] = unit
        zbuf[i, pl.ds(0, 16)] = zero
        return 0

    lax.fori_loop(0, CHUNK, _fill, 0)

    for t in range(ZB):
        pltpu.sync_copy(zbuf, agg.at[pl.ds((sid * ZB + t) * CHUNK, CHUNK)])
    plsc.subcore_barrier()

    pltpu.sync_copy(dst_hbm.at[wid], dst_v)

    def _chunk(j, _):
        pltpu.sync_copy(ones_v, agg.at[dst_v.at[j]], add=True)
        return 0

    lax.fori_loop(0, CH, _chunk, 0)
    plsc.subcore_barrier()

    for t in range(ZB):
        r0 = (sid * ZB + t) * CHUNK
        pltpu.sync_copy(agg.at[pl.ds(r0, CHUNK)], zbuf)
        pltpu.sync_copy(zbuf, out_hbm.at[cid, pl.ds(r0, CHUNK)])


@functools.partial(
    pl.kernel,
    out_type=jax.ShapeDtypeStruct((NC, N_PAD, HID), jnp.float32),
    mesh=_mesh,
    compiler_params=_sc_params,
    scratch_types=[
        pltpu.VMEM((CH, CHUNK), jnp.int32),
        pltpu.VMEM((CH, CHUNK), jnp.int32),
        pltpu.VMEM((CHUNK, HID), jnp.float32),
        pltpu.VMEM((CHUNK, HID), jnp.float32),
        pltpu.VMEM((CHUNK, HID), jnp.float32),
        pltpu.VMEM_SHARED((N_PAD, HID), jnp.float32),
        pltpu.SemaphoreType.DMA,
        pltpu.SemaphoreType.DMA,
    ],
)
def _sc_conv(src_hbm, dst_hbm, p_hbm, out_hbm, src_v, dst_v, rows0, rows1,
             zbuf, agg, sem0, sem1):
    cid = lax.axis_index("c")
    sid = lax.axis_index("s")
    wid = cid * NS + sid

    zero = jnp.zeros((16,), jnp.float32)

    def _fill(i, _):
        for c0 in range(HID // 16):
            zbuf[i, pl.ds(c0 * 16, 16)] = zero
        return 0

    lax.fori_loop(0, CHUNK, _fill, 0)

    for t in range(ZB):
        pltpu.sync_copy(zbuf, agg.at[pl.ds((sid * ZB + t) * CHUNK, CHUNK)])
    plsc.subcore_barrier()

    pltpu.sync_copy(src_hbm.at[wid], src_v)
    pltpu.sync_copy(dst_hbm.at[wid], dst_v)

    pltpu.async_copy(p_hbm.at[src_v.at[0]], rows0, sem0)

    def _pair(m, _):
        j0 = 2 * m
        pltpu.async_copy(p_hbm.at[src_v.at[j0 + 1]], rows1, sem1)
        pltpu.make_async_copy(p_hbm.at[src_v.at[j0]], rows0, sem0).wait()
        pltpu.sync_copy(rows0, agg.at[dst_v.at[j0]], add=True)
        pltpu.async_copy(p_hbm.at[src_v.at[j0 + 2]], rows0, sem0)
        pltpu.make_async_copy(p_hbm.at[src_v.at[j0 + 1]], rows1, sem1).wait()
        pltpu.sync_copy(rows1, agg.at[dst_v.at[j0 + 1]], add=True)
        return 0

    lax.fori_loop(0, CH // 2 - 1, _pair, 0)
    pltpu.async_copy(p_hbm.at[src_v.at[CH - 1]], rows1, sem1)
    pltpu.make_async_copy(p_hbm.at[src_v.at[CH - 2]], rows0, sem0).wait()
    pltpu.sync_copy(rows0, agg.at[dst_v.at[CH - 2]], add=True)
    pltpu.make_async_copy(p_hbm.at[src_v.at[CH - 1]], rows1, sem1).wait()
    pltpu.sync_copy(rows1, agg.at[dst_v.at[CH - 1]], add=True)
    plsc.subcore_barrier()

    for t in range(ZB):
        r0 = (sid * ZB + t) * CHUNK
        pltpu.sync_copy(agg.at[pl.ds(r0, CHUNK)], zbuf)
        pltpu.sync_copy(zbuf, out_hbm.at[cid, pl.ds(r0, CHUNK)])


def _dinv(degr):
    hist = degr[0, :N, 0:1] + degr[1, :N, 0:1]
    return lax.rsqrt(hist + 1.0)


def _pad_cols(g):
    return jnp.concatenate([g, jnp.zeros((N, 128 - HID), jnp.float32)],
                           axis=1)


def _tca(x_ref, gw, gb, gms, w1, degr, p_ref):
    x = x_ref[...]
    mean = jnp.sum(x, axis=0, keepdims=True) * (1.0 / N)
    xc = x - gms[...] * mean
    var = jnp.sum(xc * xc, axis=0, keepdims=True) * (1.0 / N)
    xn = gw[...] * xc * lax.rsqrt(var + EPS) + gb[...]
    g = jnp.dot(xn, w1[...], preferred_element_type=jnp.float32)
    p_ref[...] = _pad_cols(g * _dinv(degr))


def _tcmid(s_ref, p_ref, b_r, gw, gb, gms, w_r, degr, out_ref):
    dinv = _dinv(degr)
    s = s_ref[0, :N, :] + s_ref[1, :N, :]
    c = dinv * (s + p_ref[:, :HID]) + b_r[...]
    a = jnp.maximum(c, 0.0)
    mean = jnp.sum(a, axis=0, keepdims=True) * (1.0 / N)
    ac = a - gms[...] * mean
    var = jnp.sum(ac * ac, axis=0, keepdims=True) * (1.0 / N)
    h = gw[...] * ac * lax.rsqrt(var + EPS) + gb[...]
    g = jnp.dot(h, w_r[...], preferred_element_type=jnp.float32)
    out_ref[...] = _pad_cols(g * dinv)


def _tcd(s_ref, p_ref, b_r, degr, wd_r, bd_r, wo_r, bo_r, out_ref):
    dinv = _dinv(degr)
    s = s_ref[0, :N, :] + s_ref[1, :N, :]
    c = dinv * (s + p_ref[:, :HID]) + b_r[...]
    a = jnp.maximum(c, 0.0)
    pooled = jnp.sum(a, axis=0, keepdims=True) * (1.0 / N)
    d = jnp.maximum(
        jnp.dot(pooled, wd_r[...], preferred_element_type=jnp.float32)
        + bd_r[...], 0.0)
    logits = (jnp.dot(d, wo_r[...], preferred_element_type=jnp.float32)
              + bo_r[...])
    m = jnp.max(logits, axis=1, keepdims=True)
    e = jnp.exp(logits - m)
    out_ref[...] = e / jnp.sum(e, axis=1, keepdims=True)


def _tc(body, out_shape, *args):
    return pl.pallas_call(
        body, out_shape=jax.ShapeDtypeStruct(out_shape, jnp.float32))(*args)


def kernel(x, edge_index, batch, gn0_weight, gn0_bias, gn0_mean_scale,
           W1, b1, gn1_weight, gn1_bias, gn1_mean_scale, W2, b2,
           gn2_weight, gn2_bias, gn2_mean_scale, W3, b3, Wd, bd, Wo, bo):
    src = edge_index[0].astype(jnp.int32)
    dst = edge_index[1].astype(jnp.int32)
    pad = E_PAD - E
    pad_src = (jnp.arange(pad, dtype=jnp.int32) * 97) % N
    pad_dst = N + (jnp.arange(pad, dtype=jnp.int32) % (N_PAD - N))
    src_p = (2 * jnp.concatenate([src, pad_src])).reshape(NW, CH, CHUNK)
    dst_p = jnp.concatenate([dst, pad_dst]).reshape(NW, CH, CHUNK)

    row = lambda v: v.reshape(1, -1)

    deg = _sc_deg(dst_p)

    p1 = _tc(_tca, (N, 128), x, row(gn0_weight), row(gn0_bias),
             row(gn0_mean_scale), W1, deg)
    s1 = _sc_conv(src_p, dst_p, p1.reshape(2 * N, HID))
    p2 = _tc(_tcmid, (N, 128), s1, p1, row(b1), row(gn1_weight),
             row(gn1_bias), row(gn1_mean_scale), W2, deg)
    s2 = _sc_conv(src_p, dst_p, p2.reshape(2 * N, HID))
    p3 = _tc(_tcmid, (N, 128), s2, p2, row(b2), row(gn2_weight),
             row(gn2_bias), row(gn2_mean_scale), W3, deg)
    s3 = _sc_conv(src_p, dst_p, p3.reshape(2 * N, HID))
    out = _tc(_tcd, (1, C), s3, p3, row(b3), deg, Wd, row(bd), Wo, row(bo))
    return out
